# baseline (device time: 45741 ns/iter reference)
import jax
import jax.numpy as jnp
from jax import lax
from jax.experimental import pallas as pl
from jax.experimental.pallas import tpu as pltpu

N_DEV = 4
M_PER = 1024
K_PER = 1024
N = 2048


def kernel(x, w_mat, scale_x, scale_w):
    m_total, k_per = x.shape
    k_total, n = w_mat.shape

    def body(x_ref, w_ref, sx_ref, sw_ref, out_ref, xt_ref, send_sems, recv_sems):
        me = lax.axis_index("i")

        barrier_sem = pltpu.get_barrier_semaphore()
        for d in range(1, N_DEV):
            pl.semaphore_signal(
                barrier_sem, inc=1,
                device_id=((me + d) % N_DEV,),
                device_id_type=pl.DeviceIdType.MESH,
            )
        pl.semaphore_wait(barrier_sem, N_DEV - 1)

        sends = []
        for d in range(1, N_DEV):
            tgt = (me + d) % N_DEV
            rdma = pltpu.make_async_remote_copy(
                src_ref=x_ref.at[pl.ds(tgt * M_PER, M_PER), :],
                dst_ref=xt_ref.at[me],
                send_sem=send_sems.at[d],
                recv_sem=recv_sems.at[d],
                device_id=(tgt,),
                device_id_type=pl.DeviceIdType.MESH,
            )
            rdma.start()
            sends.append(rdma)

        def block_dot(a, src):
            b = w_ref[pl.ds(src * K_PER, K_PER), :]
            return lax.dot_general(
                a, b, (((1,), (0,)), ((), ())),
                preferred_element_type=jnp.int32,
            )

        acc = block_dot(x_ref[pl.ds(me * M_PER, M_PER), :], me)

        for d in (1, 3, 2):
            src = (me - d) % N_DEV
            recv = pltpu.make_async_remote_copy(
                src_ref=x_ref.at[pl.ds(0, M_PER), :],
                dst_ref=xt_ref.at[src],
                send_sem=send_sems.at[d],
                recv_sem=recv_sems.at[d],
                device_id=((me + d) % N_DEV,),
                device_id_type=pl.DeviceIdType.MESH,
            )
            recv.wait_recv()
            acc = acc + block_dot(xt_ref[src], src)

        out_ref[:, :] = acc.astype(jnp.float32) * (sx_ref[0] * sw_ref[0])

        for rdma in sends:
            rdma.wait_send()

    return pl.pallas_call(
        body,
        out_shape=jax.ShapeDtypeStruct((M_PER, n), jnp.float32),
        in_specs=[
            pl.BlockSpec(memory_space=pltpu.VMEM),
            pl.BlockSpec(memory_space=pltpu.VMEM),
            pl.BlockSpec(memory_space=pltpu.SMEM),
            pl.BlockSpec(memory_space=pltpu.SMEM),
        ],
        out_specs=pl.BlockSpec(memory_space=pltpu.VMEM),
        scratch_shapes=[
            pltpu.VMEM((N_DEV, M_PER, K_PER), jnp.int8),
            pltpu.SemaphoreType.DMA((N_DEV,)),
            pltpu.SemaphoreType.DMA((N_DEV,)),
        ],
        compiler_params=pltpu.CompilerParams(collective_id=0),
    )(x, w_mat, scale_x, scale_w)


# device time: 40790 ns/iter; 1.1214x vs baseline; 1.1214x over previous
import jax
import jax.numpy as jnp
from jax import lax
from jax.experimental import pallas as pl
from jax.experimental.pallas import tpu as pltpu

N_DEV = 4
M_PER = 1024
K_PER = 1024
HALF = 512
QUAR = 256


def kernel(x, w_mat, scale_x, scale_w):
    m_total, k_per = x.shape
    k_total, n = w_mat.shape

    def body(x_ref, w_ref, sx_ref, sw_ref, out_hbm, xt_ref, out_v,
             send_sems, recv_sems, out_sems):
        me = lax.axis_index("i")

        barrier_sem = pltpu.get_barrier_semaphore()
        for d in range(1, N_DEV):
            pl.semaphore_signal(
                barrier_sem, inc=1,
                device_id=((me + d) % N_DEV,),
                device_id_type=pl.DeviceIdType.MESH,
            )
        pl.semaphore_wait(barrier_sem, N_DEV - 1)

        CHUNKS = (
            [(1, c, c * HALF, HALF) for c in range(2)]
            + [(3, c, c * HALF, HALF) for c in range(2)]
            + [(2, q, q * QUAR, QUAR) for q in range(4)]
        )

        def make_rdma(d, slot, off, rows, sender):
            tgt = (me + d) % N_DEV
            src_dev = (me - d) % N_DEV
            slot_dev = me if sender else src_dev
            return pltpu.make_async_remote_copy(
                src_ref=x_ref.at[pl.ds(tgt * M_PER + off, rows), :],
                dst_ref=xt_ref.at[slot_dev, pl.ds(off, rows), :],
                send_sem=send_sems.at[d, slot],
                recv_sem=recv_sems.at[d, slot],
                device_id=(tgt,),
                device_id_type=pl.DeviceIdType.MESH,
            )

        sends = []
        for d, slot, off, rows in CHUNKS:
            rdma = make_rdma(d, slot, off, rows, sender=True)
            rdma.start()
            sends.append(rdma)


        def block_dot(a, src):
            b = w_ref[pl.ds(src * K_PER, K_PER), :]
            return lax.dot_general(
                a, b, (((1,), (0,)), ((), ())),
                preferred_element_type=jnp.int32,
            )

        def wait_chunk(d, slot, off, rows):
            make_rdma(d, slot, off, rows, sender=False).wait_recv()
            return (me - d) % N_DEV

        scale = sx_ref[0] * sw_ref[0]

        acc = [None] * 4

        def add_half(c, val):
            for h in range(2):
                q = 2 * c + h
                part = val[h * QUAR:(h + 1) * QUAR, :]
                acc[q] = part if acc[q] is None else acc[q] + part

        out_cp = []

        def finish_quarter(q):
            off = q * QUAR
            out_v[pl.ds(off, QUAR), :] = acc[q].astype(jnp.float32) * scale
            cp = pltpu.make_async_copy(
                out_v.at[pl.ds(off, QUAR), :],
                out_hbm.at[pl.ds(off, QUAR), :],
                out_sems.at[q],
            )
            cp.start()
            out_cp.append(cp)

        for c in range(2):
            add_half(c, block_dot(
                x_ref[pl.ds(me * M_PER + c * HALF, HALF), :], me))

        for d in (1, 3):
            src = wait_chunk(d, 0, 0, HALF)
            add_half(0, block_dot(xt_ref[src, pl.ds(0, HALF), :], src))

        for q in range(2):
            src = wait_chunk(2, q, q * QUAR, QUAR)
            acc[q] = acc[q] + block_dot(
                xt_ref[src, pl.ds(q * QUAR, QUAR), :], src)
            finish_quarter(q)

        for d in (1, 3):
            src = wait_chunk(d, 1, HALF, HALF)
            add_half(1, block_dot(xt_ref[src, pl.ds(HALF, HALF), :], src))

        for q in range(2, 4):
            src = wait_chunk(2, q, q * QUAR, QUAR)
            acc[q] = acc[q] + block_dot(
                xt_ref[src, pl.ds(q * QUAR, QUAR), :], src)
            finish_quarter(q)

        for cp in out_cp:
            cp.wait()
        for rdma in sends:
            rdma.wait_send()

    return pl.pallas_call(
        body,
        out_shape=jax.ShapeDtypeStruct((M_PER, n), jnp.float32),
        in_specs=[
            pl.BlockSpec(memory_space=pltpu.VMEM),
            pl.BlockSpec(memory_space=pltpu.VMEM),
            pl.BlockSpec(memory_space=pltpu.SMEM),
            pl.BlockSpec(memory_space=pltpu.SMEM),
        ],
        out_specs=pl.BlockSpec(memory_space=pl.ANY),
        scratch_shapes=[
            pltpu.VMEM((N_DEV, M_PER, K_PER), jnp.int8),
            pltpu.VMEM((M_PER, n), jnp.float32),
            pltpu.SemaphoreType.DMA((N_DEV, 4)),
            pltpu.SemaphoreType.DMA((N_DEV, 4)),
            pltpu.SemaphoreType.DMA((4,)),
        ],
        compiler_params=pltpu.CompilerParams(collective_id=0),
    )(x, w_mat, scale_x, scale_w)
